# baseline (speedup 1.0000x reference)
import jax
import jax.numpy as jnp
from jax import lax
from jax.experimental import pallas as pl
from jax.experimental.pallas import tpu as pltpu
from jax.experimental.pallas import tpu_sc as plsc

_B, _S, _D = 4, 8192, 768
_NC, _NS = 2, 16
_NW = _NC * _NS
_S_PER_W = _S // _NW
_CS = 32
_NCHUNK = _S_PER_W // _CS
_T = _NCHUNK * _B
_LANES = 16
_GROUPS = _D // _LANES


def _sc_body(in_hbm, tbl_hbm, out_hbm,
             d0, d1, d2, t0, t1,
             ls0, ls1, ls2, ss0, ss1, ss2, ts0, ts1):
    dbufs = (d0, d1, d2)
    tbls = (t0, t1)
    lsems = (ls0, ls1, ls2)
    ssems = (ss0, ss1, ss2)
    tsems = (ts0, ts1)

    wid = lax.axis_index("s") * _NC + lax.axis_index("c")
    s_base = wid * _S_PER_W

    def issue_load(t):
        c, b, i = t // _B, t % _B, t % 3
        pltpu.async_copy(in_hbm.at[b, pl.ds(s_base + c * _CS, _CS)],
                         dbufs[i], lsems[i])

    def issue_store(t):
        c, b, i = t // _B, t % _B, t % 3
        pltpu.async_copy(dbufs[i], out_hbm.at[b, pl.ds(s_base + c * _CS, _CS)],
                         ssems[i])

    def issue_tbl(c):
        j = c % 2
        pltpu.async_copy(tbl_hbm.at[pl.ds(s_base + c * _CS, _CS)],
                         tbls[j], tsems[j])

    def wait_load(i):
        pltpu.make_async_copy(in_hbm.at[0, pl.ds(s_base, _CS)], dbufs[i],
                              lsems[i]).wait()

    def wait_store(i):
        pltpu.make_async_copy(dbufs[i], out_hbm.at[0, pl.ds(s_base, _CS)],
                              ssems[i]).wait()

    def wait_tbl(j):
        pltpu.make_async_copy(tbl_hbm.at[pl.ds(s_base, _CS)], tbls[j],
                              tsems[j]).wait()

    def compute(i, j):
        def row(r, carry):
            for g in range(_GROUPS):
                plsc.addupdate(
                    dbufs[i].at[r, pl.ds(g * _LANES, _LANES)],
                    tbls[j][r, pl.ds(g * _LANES, _LANES)],
                )
            return carry
        lax.fori_loop(0, _CS, row, 0)

    issue_tbl(0)
    issue_tbl(1)
    for t in range(3):
        issue_load(t)

    for t in range(_T):
        c, b, i = t // _B, t % _B, t % 3
        if b == 0:
            wait_tbl(c % 2)
        wait_load(i)
        compute(i, c % 2)
        issue_store(t)
        if b == 3 and c + 2 < _NCHUNK:
            issue_tbl(c + 2)
        if t >= 1:
            wait_store((t - 1) % 3)
            if t + 2 < _T:
                issue_load(t + 2)

    wait_store((_T - 1) % 3)


_sc_call = pl.kernel(
    _sc_body,
    out_type=jax.ShapeDtypeStruct((_B, _S, _D), jnp.float32),
    mesh=plsc.VectorSubcoreMesh(core_axis_name="c", subcore_axis_name="s"),
    scratch_types=(
        [pltpu.VMEM((_CS, _D), jnp.float32)] * 3
        + [pltpu.VMEM((_CS, _D), jnp.float32)] * 2
        + [pltpu.SemaphoreType.DMA] * 8
    ),
)


def kernel(inputs, pos_table):
    return _sc_call(inputs, pos_table)
